# baseline (speedup 1.0000x reference)
import functools

import jax
import jax.numpy as jnp
import numpy as np
from jax.experimental import pallas as pl
from jax.experimental.pallas import tpu as pltpu

_M0 = 0.01
_LIMS = [[-1.0, 1.0], [-1.0, 1.0]]
_SIZE = [128, 128]

_LEVELS = 32

def _oe_merge(lo, hi, r):
    step = r * 2
    if step < hi - lo:
        yield from _oe_merge(lo, hi, step)
        yield from _oe_merge(lo + r, hi, step)
        for i in range(lo + r, hi - r + 1, step):
            yield (i, i + r)
    else:
        yield (lo, lo + r)


def _oe_sort(lo, hi):
    if hi - lo >= 1:
        mid = lo + (hi - lo) // 2
        yield from _oe_sort(lo, mid)
        yield from _oe_sort(mid + 1, hi)
        yield from _oe_merge(lo, hi, 1)


_SORTNET = list(_oe_sort(0, _LEVELS - 1))


def _dtm_body(gx_ref, gy_ref, x0_ref, x1_ref, out_ref, *, k, bound, cum_w):
    gx = gx_ref[...]
    gy = gy_ref[...]
    x0 = x0_ref[0:1, :]
    x1 = x1_ref[0:1, :]

    n = x0.shape[1]
    w = n // _LEVELS

    vals = []
    for t in range(_LEVELS):
        sl = slice(t * w, (t + 1) * w)
        d2 = (gx - x0[:, sl]) ** 2 + (gy - x1[:, sl]) ** 2
        vals.append(d2.astype(jnp.bfloat16))

    for a, b in _SORTNET:
        lo = jnp.minimum(vals[a], vals[b])
        hi = jnp.maximum(vals[a], vals[b])
        vals[a], vals[b] = lo, hi

    r = gx.shape[0]
    inf = jnp.bfloat16(jnp.inf)

    one = jnp.bfloat16(1.0)
    f = vals[0]
    h = jnp.zeros_like(f)
    ones_col = jnp.ones((w, 1), jnp.bfloat16)

    s = jnp.zeros((r, 1), jnp.float32)
    kth = jnp.zeros((r, 1), jnp.float32)
    rem = jnp.full((r, 1), jnp.float32(k))

    for t in range(k):
        m = jnp.min(f, axis=1, keepdims=True)
        hit = f == m
        c = jax.lax.dot_general(
            jnp.where(hit, one, jnp.bfloat16(0.0)), ones_col,
            (((1,), (0,)), ((), ())),
            preferred_element_type=jnp.float32)
        take = jnp.minimum(c, rem)
        m32 = m.astype(jnp.float32)
        s = s + take * jnp.minimum(m32, jnp.float32(3.3e38))
        kth = jnp.maximum(kth, jnp.where(rem > 0, m32, jnp.float32(0.0)))
        rem = rem - take
        if t < k - 1:
            h = h + jnp.where(hit, one, jnp.bfloat16(0.0))
            imax = min(t + 1, _LEVELS - 1)
            upd = jnp.full_like(f, inf)
            for i in range(imax, 0, -1):
                upd = jnp.where(h == jnp.bfloat16(i), vals[i], upd)
            f = jnp.where(hit, upd, f)

    val = jnp.maximum(s + kth * jnp.float32(bound - cum_w), jnp.float32(0.0))
    out_ref[...] = jnp.sqrt(val * jnp.float32(1.0 / bound))


@jax.jit
def kernel(x):
    n = x.shape[-2]
    bound = _M0 * n
    k = int(np.ceil(bound))
    cum_w = float(np.ceil(bound))

    x_seq = jnp.linspace(_LIMS[0][0], _LIMS[0][1], _SIZE[0])
    y_seq = jnp.linspace(_LIMS[1][1], _LIMS[1][0], _SIZE[1])
    x_coord, y_coord = jnp.meshgrid(x_seq, y_seq, indexing='xy')
    gx = x_coord.reshape(-1, 1).astype(jnp.float32)
    gy = y_coord.reshape(-1, 1).astype(jnp.float32)

    hw = _SIZE[0] * _SIZE[1]
    rows = 1024

    x0 = jnp.broadcast_to(x[:, 0].reshape(1, n), (8, n))
    x1 = jnp.broadcast_to(x[:, 1].reshape(1, n), (8, n))

    out = pl.pallas_call(
        functools.partial(_dtm_body, k=k, bound=bound, cum_w=cum_w),
        grid=(hw // rows,),
        in_specs=[
            pl.BlockSpec((rows, 1), lambda i: (i, 0)),
            pl.BlockSpec((rows, 1), lambda i: (i, 0)),
            pl.BlockSpec((8, n), lambda i: (0, 0)),
            pl.BlockSpec((8, n), lambda i: (0, 0)),
        ],
        out_specs=pl.BlockSpec((rows, 1), lambda i: (i, 0)),
        out_shape=jax.ShapeDtypeStruct((hw, 1), jnp.float32),
    )(gx, gy, x0, x1)

    return out.reshape(_SIZE[0], _SIZE[1])


# speedup vs baseline: 1.2491x; 1.2491x over previous
import functools

import jax
import jax.numpy as jnp
import numpy as np
from jax.experimental import pallas as pl
from jax.experimental.pallas import tpu as pltpu

_M0 = 0.01
_LIMS = [[-1.0, 1.0], [-1.0, 1.0]]
_SIZE = [128, 128]

_LEVELS = 16

def _oe_merge(lo, hi, r):
    step = r * 2
    if step < hi - lo:
        yield from _oe_merge(lo, hi, step)
        yield from _oe_merge(lo + r, hi, step)
        for i in range(lo + r, hi - r + 1, step):
            yield (i, i + r)
    else:
        yield (lo, lo + r)


def _oe_sort(lo, hi):
    if hi - lo >= 1:
        mid = lo + (hi - lo) // 2
        yield from _oe_sort(lo, mid)
        yield from _oe_sort(mid + 1, hi)
        yield from _oe_merge(lo, hi, 1)


_SORTNET = list(_oe_sort(0, _LEVELS - 1))


def _dtm_body(gx_ref, gy_ref, x0_ref, x1_ref, out_ref, *, k, bound, cum_w):
    gx = gx_ref[...]
    gy = gy_ref[...]
    x0 = x0_ref[0:1, :]
    x1 = x1_ref[0:1, :]

    n = x0.shape[1]
    w = n // _LEVELS

    vals = []
    for t in range(_LEVELS):
        sl = slice(t * w, (t + 1) * w)
        d2 = (gx - x0[:, sl]) ** 2 + (gy - x1[:, sl]) ** 2
        vals.append(d2.astype(jnp.bfloat16))

    for a, b in _SORTNET:
        lo = jnp.minimum(vals[a], vals[b])
        hi = jnp.maximum(vals[a], vals[b])
        vals[a], vals[b] = lo, hi

    r = gx.shape[0]
    inf = jnp.bfloat16(jnp.inf)

    one = jnp.bfloat16(1.0)
    f = vals[0]
    h = jnp.zeros_like(f)
    ones_col = jnp.ones((w, 1), jnp.bfloat16)

    s = jnp.zeros((r, 1), jnp.float32)
    kth = jnp.zeros((r, 1), jnp.float32)
    rem = jnp.full((r, 1), jnp.float32(k))

    for t in range(k):
        m = jnp.min(f, axis=1, keepdims=True)
        hit = f == m
        c = jax.lax.dot_general(
            jnp.where(hit, one, jnp.bfloat16(0.0)), ones_col,
            (((1,), (0,)), ((), ())),
            preferred_element_type=jnp.float32)
        take = jnp.minimum(c, rem)
        m32 = m.astype(jnp.float32)
        s = s + take * jnp.minimum(m32, jnp.float32(3.3e38))
        kth = jnp.maximum(kth, jnp.where(rem > 0, m32, jnp.float32(0.0)))
        rem = rem - take
        if t < k - 1:
            h = h + jnp.where(hit, one, jnp.bfloat16(0.0))
            imax = min(t + 1, _LEVELS - 1)
            upd = jnp.full_like(f, inf)
            for i in range(imax, 0, -1):
                upd = jnp.where(h == jnp.bfloat16(i), vals[i], upd)
            f = jnp.where(hit, upd, f)

    val = jnp.maximum(s + kth * jnp.float32(bound - cum_w), jnp.float32(0.0))
    out_ref[...] = jnp.sqrt(val * jnp.float32(1.0 / bound))


@jax.jit
def kernel(x):
    n = x.shape[-2]
    bound = _M0 * n
    k = int(np.ceil(bound))
    cum_w = float(np.ceil(bound))

    x_seq = jnp.linspace(_LIMS[0][0], _LIMS[0][1], _SIZE[0])
    y_seq = jnp.linspace(_LIMS[1][1], _LIMS[1][0], _SIZE[1])
    x_coord, y_coord = jnp.meshgrid(x_seq, y_seq, indexing='xy')
    gx = x_coord.reshape(-1, 1).astype(jnp.float32)
    gy = y_coord.reshape(-1, 1).astype(jnp.float32)

    hw = _SIZE[0] * _SIZE[1]
    rows = 512

    x0 = jnp.broadcast_to(x[:, 0].reshape(1, n), (8, n))
    x1 = jnp.broadcast_to(x[:, 1].reshape(1, n), (8, n))

    out = pl.pallas_call(
        functools.partial(_dtm_body, k=k, bound=bound, cum_w=cum_w),
        grid=(hw // rows,),
        in_specs=[
            pl.BlockSpec((rows, 1), lambda i: (i, 0)),
            pl.BlockSpec((rows, 1), lambda i: (i, 0)),
            pl.BlockSpec((8, n), lambda i: (0, 0)),
            pl.BlockSpec((8, n), lambda i: (0, 0)),
        ],
        out_specs=pl.BlockSpec((rows, 1), lambda i: (i, 0)),
        out_shape=jax.ShapeDtypeStruct((hw, 1), jnp.float32),
    )(gx, gy, x0, x1)

    return out.reshape(_SIZE[0], _SIZE[1])
